# baseline (speedup 1.0000x reference)
import jax
import jax.numpy as jnp
import numpy as np
from jax import lax
from jax.experimental import pallas as pl
from jax.experimental.pallas import tpu as pltpu
from jax.experimental.pallas import tpu_sc as plsc

VOCAB = 100000
D = 128
B = 1024
L = 200

_NUM_CORES = 2
_NUM_SUBCORES = 16
_NW = _NUM_CORES * _NUM_SUBCORES
_ROWS = B * L
_ROWS_PER_W = _ROWS // _NW
_CHUNK = 40
_NCHUNK = _ROWS_PER_W // _CHUNK
_NBUF = 4
_LPAD = _ROWS_PER_W + 16


def _sin_pe(max_len, d_model):
    pos = np.arange(max_len, dtype=np.float32)[:, None]
    div = np.exp(
        np.arange(0, d_model, 2, dtype=np.float32) * -(np.log(10000.0) / d_model)
    )
    pe = np.zeros((max_len, d_model), dtype=np.float32)
    pe[:, 0::2] = np.sin(pos * div)
    pe[:, 1::2] = np.cos(pos * div)
    return pe


_PE = _sin_pe(L, D)


def _embed_kernel(tok_idx_hbm, labels_hbm, tok_table_hbm, combo_hbm, out_hbm,
                  idx_t, labels, combo_v, rows0, rows1, rows2, rows3,
                  gsem0, gsem1, gsem2, gsem3, osem0, osem1, osem2, osem3):
    wid = lax.axis_index("s") * _NUM_CORES + lax.axis_index("c")
    base = wid * _ROWS_PER_W
    rows = (rows0, rows1)
    outs = (rows2, rows3)
    gsems = (gsem0, gsem1)
    osems = (osem0, osem1)

    pltpu.sync_copy(tok_idx_hbm.at[pl.ds(base, _ROWS_PER_W)], idx_t)
    pltpu.sync_copy(labels_hbm.at[pl.ds(base, _ROWS_PER_W)],
                    labels.at[pl.ds(0, _ROWS_PER_W)])
    pltpu.sync_copy(combo_hbm, combo_v)

    lanes = lax.iota(jnp.int32, 16)
    full_mask = lanes < 16
    tail_mask = lanes < (_CHUNK % 16 if _CHUNK % 16 else 16)

    def start_gather(g, b):
        pltpu.async_copy(
            tok_table_hbm.at[idx_t.at[pl.ds(g * _CHUNK, _CHUNK)]],
            rows[b], gsems[b])

    def wait_gather(b):
        pltpu.make_async_copy(
            tok_table_hbm.at[idx_t.at[pl.ds(0, _CHUNK)]],
            rows[b], gsems[b]).wait()

    def wait_out(b):
        pltpu.make_async_copy(
            outs[b], out_hbm.at[pl.ds(0, _CHUNK)], osems[b]).wait()

    def start_out(g, b):
        pltpu.async_copy(
            outs[b], out_hbm.at[pl.ds(base + g * _CHUNK, _CHUNK)], osems[b])

    def compute(g, b):
        buf = rows[b]
        obuf = outs[b]
        l0 = lax.rem(g * _CHUNK, L)
        for r0 in range(0, _CHUNK, 16):
            nj = min(16, _CHUNK - r0)
            mask = full_mask if nj == 16 else tail_mask
            rowidx = lanes + r0
            cvec = labels[pl.ds(g * _CHUNK + r0, 16)]
            crow = cvec * L + (l0 + r0) + lanes

            def col_body(c, _):
                cc = jnp.full((16,), c, dtype=jnp.int32)
                tok = plsc.load_gather(buf, [rowidx, cc], mask=mask)
                cmb = plsc.load_gather(combo_v, [crow, cc], mask=mask)
                plsc.store_scatter(obuf, [rowidx, cc], tok + cmb, mask=mask)
                return ()

            lax.fori_loop(0, D, col_body, (), unroll=4)

    start_gather(0, 0)

    def pair_body(p, _):
        for b in range(2):
            g = p * 2 + b
            b1 = 1 - b
            if b == 0:
                start_gather(g + 1, b1)
            else:
                @pl.when(p < _NCHUNK // 2 - 1)
                def _():
                    start_gather(g + 1, b1)
            wait_gather(b)

            @pl.when(p >= 1)
            def _():
                wait_out(b)
            compute(g, b)
            start_out(g, b)
        return ()

    lax.fori_loop(0, _NCHUNK // 2, pair_body, ())
    for b in range(2):
        wait_out(b)


@jax.jit
def kernel(sequence, segment_label, token_table, seg_table):
    tok_idx = sequence.reshape(-1).astype(jnp.int32)
    labels = segment_label.reshape(-1).astype(jnp.int32)
    combo = (seg_table[:, None, :] + jnp.asarray(_PE)[None, :, :]).reshape(3 * L, D)

    mesh = plsc.VectorSubcoreMesh(core_axis_name="c", subcore_axis_name="s")
    run = pl.kernel(
        _embed_kernel,
        mesh=mesh,
        compiler_params=pltpu.CompilerParams(needs_layout_passes=False),
        out_type=jax.ShapeDtypeStruct((_ROWS, D), jnp.float32),
        scratch_types=[
            pltpu.VMEM((_ROWS_PER_W,), jnp.int32),
            pltpu.VMEM((_LPAD,), jnp.int32),
            pltpu.VMEM((3 * L, D), jnp.float32),
            pltpu.VMEM((_CHUNK, D), jnp.float32),
            pltpu.VMEM((_CHUNK, D), jnp.float32),
            pltpu.VMEM((_CHUNK, D), jnp.float32),
            pltpu.VMEM((_CHUNK, D), jnp.float32),
            pltpu.SemaphoreType.DMA,
            pltpu.SemaphoreType.DMA,
            pltpu.SemaphoreType.DMA,
            pltpu.SemaphoreType.DMA,
            pltpu.SemaphoreType.DMA,
            pltpu.SemaphoreType.DMA,
            pltpu.SemaphoreType.DMA,
            pltpu.SemaphoreType.DMA,
        ],
    )
    out = run(tok_idx, labels, token_table, combo)
    return out.reshape(B, L, D)


# speedup vs baseline: 4.0794x; 4.0794x over previous
import jax
import jax.numpy as jnp
import numpy as np
from jax import lax
from jax.experimental import pallas as pl
from jax.experimental.pallas import tpu as pltpu
from jax.experimental.pallas import tpu_sc as plsc

VOCAB = 100000
D = 128
B = 1024
L = 200

_NUM_CORES = 2
_NUM_SUBCORES = 16
_NW = _NUM_CORES * _NUM_SUBCORES
_ROWS = B * L
_ROWS_PER_W = _ROWS // _NW
_CHUNK = 40
_NCHUNK = _ROWS_PER_W // _CHUNK
_NBUF = 4
_LPAD = _ROWS_PER_W + 16


def _sin_pe(max_len, d_model):
    pos = np.arange(max_len, dtype=np.float32)[:, None]
    div = np.exp(
        np.arange(0, d_model, 2, dtype=np.float32) * -(np.log(10000.0) / d_model)
    )
    pe = np.zeros((max_len, d_model), dtype=np.float32)
    pe[:, 0::2] = np.sin(pos * div)
    pe[:, 1::2] = np.cos(pos * div)
    return pe


_PE = _sin_pe(L, D)


def _embed_kernel(tok_idx_hbm, labels_hbm, tok_table_hbm, combo_hbm, out_hbm,
                  idx_t, labels, combo_v, rows0, rows1, rows2, rows3,
                  gsem0, gsem1, gsem2, gsem3, osem0, osem1, osem2, osem3):
    wid = lax.axis_index("s") * _NUM_CORES + lax.axis_index("c")
    base = wid * _ROWS_PER_W
    rows = (rows0, rows1)
    outs = (rows2, rows3)
    gsems = (gsem0, gsem1)
    osems = (osem0, osem1)

    pltpu.sync_copy(tok_idx_hbm.at[pl.ds(base, _ROWS_PER_W)], idx_t)
    pltpu.sync_copy(labels_hbm.at[pl.ds(base, _ROWS_PER_W)],
                    labels.at[pl.ds(0, _ROWS_PER_W)])
    pltpu.sync_copy(combo_hbm, combo_v)

    def start_gather(g, b):
        pltpu.async_copy(
            tok_table_hbm.at[idx_t.at[pl.ds(g * _CHUNK, _CHUNK)]],
            rows[b], gsems[b])

    def wait_gather(b):
        pltpu.make_async_copy(
            tok_table_hbm.at[idx_t.at[pl.ds(0, _CHUNK)]],
            rows[b], gsems[b]).wait()

    def wait_out(b):
        pltpu.make_async_copy(
            outs[b], out_hbm.at[pl.ds(0, _CHUNK)], osems[b]).wait()

    def start_out(g, b):
        pltpu.async_copy(
            outs[b], out_hbm.at[pl.ds(base + g * _CHUNK, _CHUNK)], osems[b])

    def compute(g, b):
        buf = rows[b]
        obuf = outs[b]
        l0 = lax.rem(g * _CHUNK, L)
        for r0 in range(0, _CHUNK, 16):
            nj = min(16, _CHUNK - r0)
            cvec = labels[pl.ds(g * _CHUNK + r0, 16)]
            for j in range(nj):
                crow = cvec[j] * L + (l0 + (r0 + j))
                for k in range(D // 16):
                    sk = pl.ds(k * 16, 16)
                    obuf[r0 + j, sk] = buf[r0 + j, sk] + combo_v[crow, sk]

    start_gather(0, 0)

    def pair_body(p, _):
        for b in range(2):
            g = p * 2 + b
            b1 = 1 - b
            if b == 0:
                start_gather(g + 1, b1)
            else:
                @pl.when(p < _NCHUNK // 2 - 1)
                def _():
                    start_gather(g + 1, b1)
            wait_gather(b)

            @pl.when(p >= 1)
            def _():
                wait_out(b)
            compute(g, b)
            start_out(g, b)
        return ()

    lax.fori_loop(0, _NCHUNK // 2, pair_body, ())
    for b in range(2):
        wait_out(b)


@jax.jit
def kernel(sequence, segment_label, token_table, seg_table):
    tok_idx = sequence.reshape(-1).astype(jnp.int32)
    labels = segment_label.reshape(-1).astype(jnp.int32)
    combo = (seg_table[:, None, :] + jnp.asarray(_PE)[None, :, :]).reshape(3 * L, D)

    mesh = plsc.VectorSubcoreMesh(core_axis_name="c", subcore_axis_name="s")
    run = pl.kernel(
        _embed_kernel,
        mesh=mesh,
        compiler_params=pltpu.CompilerParams(needs_layout_passes=False),
        out_type=jax.ShapeDtypeStruct((_ROWS, D), jnp.float32),
        scratch_types=[
            pltpu.VMEM((_ROWS_PER_W,), jnp.int32),
            pltpu.VMEM((_LPAD,), jnp.int32),
            pltpu.VMEM((3 * L, D), jnp.float32),
            pltpu.VMEM((_CHUNK, D), jnp.float32),
            pltpu.VMEM((_CHUNK, D), jnp.float32),
            pltpu.VMEM((_CHUNK, D), jnp.float32),
            pltpu.VMEM((_CHUNK, D), jnp.float32),
            pltpu.SemaphoreType.DMA,
            pltpu.SemaphoreType.DMA,
            pltpu.SemaphoreType.DMA,
            pltpu.SemaphoreType.DMA,
            pltpu.SemaphoreType.DMA,
            pltpu.SemaphoreType.DMA,
            pltpu.SemaphoreType.DMA,
            pltpu.SemaphoreType.DMA,
        ],
    )
    out = run(tok_idx, labels, token_table, combo)
    return out.reshape(B, L, D)


# speedup vs baseline: 7.8442x; 1.9229x over previous
import jax
import jax.numpy as jnp
import numpy as np
from jax import lax
from jax.experimental import pallas as pl
from jax.experimental.pallas import tpu as pltpu
from jax.experimental.pallas import tpu_sc as plsc

VOCAB = 100000
D = 128
B = 1024
L = 200

_NUM_CORES = 2
_NUM_SUBCORES = 16
_NW = _NUM_CORES * _NUM_SUBCORES
_ROWS = B * L
_ROWS_PER_W = _ROWS // _NW
_CHUNK = 128
_NCHUNK = _ROWS_PER_W // _CHUNK


def _sin_pe(max_len, d_model):
    pos = np.arange(max_len, dtype=np.float32)[:, None]
    div = np.exp(
        np.arange(0, d_model, 2, dtype=np.float32) * -(np.log(10000.0) / d_model)
    )
    pe = np.zeros((max_len, d_model), dtype=np.float32)
    pe[:, 0::2] = np.sin(pos * div)
    pe[:, 1::2] = np.cos(pos * div)
    return pe


_PE = _sin_pe(L, D)


def _embed_kernel(tok_idx_hbm, combo_idx_hbm, tok_table_hbm, combo_hbm, out_hbm,
                  idx_t, idx_c, tok0, tok1, cmb0, cmb1, out0, out1,
                  tsem0, tsem1, csem0, csem1, osem0, osem1):
    wid = lax.axis_index("s") * _NUM_CORES + lax.axis_index("c")
    base = wid * _ROWS_PER_W
    toks = (tok0, tok1)
    cmbs = (cmb0, cmb1)
    outs = (out0, out1)
    tsems = (tsem0, tsem1)
    csems = (csem0, csem1)
    osems = (osem0, osem1)

    pltpu.sync_copy(tok_idx_hbm.at[pl.ds(base, _ROWS_PER_W)], idx_t)
    pltpu.sync_copy(combo_idx_hbm.at[pl.ds(base, _ROWS_PER_W)], idx_c)

    def start_gathers(g, b):
        pltpu.async_copy(
            tok_table_hbm.at[idx_t.at[pl.ds(g * _CHUNK, _CHUNK)]],
            toks[b], tsems[b])
        pltpu.async_copy(
            combo_hbm.at[idx_c.at[pl.ds(g * _CHUNK, _CHUNK)]],
            cmbs[b], csems[b])

    def wait_gathers(b):
        pltpu.make_async_copy(
            tok_table_hbm.at[idx_t.at[pl.ds(0, _CHUNK)]],
            toks[b], tsems[b]).wait()
        pltpu.make_async_copy(
            combo_hbm.at[idx_c.at[pl.ds(0, _CHUNK)]],
            cmbs[b], csems[b]).wait()

    def wait_out(b):
        pltpu.make_async_copy(
            outs[b], out_hbm.at[pl.ds(0, _CHUNK)], osems[b]).wait()

    def start_out(g, b):
        pltpu.async_copy(
            outs[b], out_hbm.at[pl.ds(base + g * _CHUNK, _CHUNK)], osems[b])

    def compute(b):
        tbuf, cbuf, obuf = toks[b], cmbs[b], outs[b]

        def row_body(j, _):
            for k in range(D // 16):
                sk = pl.ds(k * 16, 16)
                obuf[j, sk] = tbuf[j, sk] + cbuf[j, sk]
            return ()

        lax.fori_loop(0, _CHUNK, row_body, ())

    start_gathers(0, 0)

    def pair_body(p, _):
        for b in range(2):
            g = p * 2 + b
            b1 = 1 - b
            if b == 0:
                start_gathers(g + 1, b1)
            else:
                @pl.when(p < _NCHUNK // 2 - 1)
                def _():
                    start_gathers(g + 1, b1)
            wait_gathers(b)

            @pl.when(p >= 1)
            def _():
                wait_out(b)
            compute(b)
            start_out(g, b)
        return ()

    lax.fori_loop(0, _NCHUNK // 2, pair_body, ())
    for b in range(2):
        wait_out(b)


@jax.jit
def kernel(sequence, segment_label, token_table, seg_table):
    tok_idx = sequence.reshape(-1).astype(jnp.int32)
    pos = jnp.arange(L, dtype=jnp.int32)
    combo_idx = (segment_label.astype(jnp.int32) * L + pos[None, :]).reshape(-1)
    combo = (seg_table[:, None, :] + jnp.asarray(_PE)[None, :, :]).reshape(3 * L, D)

    mesh = plsc.VectorSubcoreMesh(core_axis_name="c", subcore_axis_name="s")
    run = pl.kernel(
        _embed_kernel,
        mesh=mesh,
        out_type=jax.ShapeDtypeStruct((_ROWS, D), jnp.float32),
        scratch_types=[
            pltpu.VMEM((_ROWS_PER_W,), jnp.int32),
            pltpu.VMEM((_ROWS_PER_W,), jnp.int32),
            pltpu.VMEM((_CHUNK, D), jnp.float32),
            pltpu.VMEM((_CHUNK, D), jnp.float32),
            pltpu.VMEM((_CHUNK, D), jnp.float32),
            pltpu.VMEM((_CHUNK, D), jnp.float32),
            pltpu.VMEM((_CHUNK, D), jnp.float32),
            pltpu.VMEM((_CHUNK, D), jnp.float32),
            pltpu.SemaphoreType.DMA,
            pltpu.SemaphoreType.DMA,
            pltpu.SemaphoreType.DMA,
            pltpu.SemaphoreType.DMA,
            pltpu.SemaphoreType.DMA,
            pltpu.SemaphoreType.DMA,
        ],
    )
    out = run(tok_idx, combo_idx, token_table, combo)
    return out.reshape(B, L, D)


# speedup vs baseline: 8.0750x; 1.0294x over previous
import jax
import jax.numpy as jnp
import numpy as np
from jax import lax
from jax.experimental import pallas as pl
from jax.experimental.pallas import tpu as pltpu
from jax.experimental.pallas import tpu_sc as plsc

VOCAB = 100000
D = 128
B = 1024
L = 200

_NUM_CORES = 2
_NUM_SUBCORES = 16
_NW = _NUM_CORES * _NUM_SUBCORES
_ROWS = B * L
_ROWS_PER_W = _ROWS // _NW
_CHUNK = 128
_NCHUNK = _ROWS_PER_W // _CHUNK


def _sin_pe(max_len, d_model):
    pos = np.arange(max_len, dtype=np.float32)[:, None]
    div = np.exp(
        np.arange(0, d_model, 2, dtype=np.float32) * -(np.log(10000.0) / d_model)
    )
    pe = np.zeros((max_len, d_model), dtype=np.float32)
    pe[:, 0::2] = np.sin(pos * div)
    pe[:, 1::2] = np.cos(pos * div)
    return pe


_PE = _sin_pe(L, D)


def _embed_kernel(tok_idx_hbm, combo_idx_hbm, tok_table_hbm, combo_hbm, out_hbm,
                  idx_t, idx_c, tok0, tok1, cmb0, cmb1, out0, out1,
                  tsem0, tsem1, csem0, csem1, osem0, osem1):
    wid = lax.axis_index("s") * _NUM_CORES + lax.axis_index("c")
    base = wid * _ROWS_PER_W
    toks = (tok0, tok1)
    cmbs = (cmb0, cmb1)
    outs = (out0, out1)
    tsems = (tsem0, tsem1)
    csems = (csem0, csem1)
    osems = (osem0, osem1)

    pltpu.sync_copy(tok_idx_hbm.at[pl.ds(base, _ROWS_PER_W)], idx_t)
    pltpu.sync_copy(combo_idx_hbm.at[pl.ds(base, _ROWS_PER_W)], idx_c)

    def start_gathers(g, b):
        pltpu.async_copy(
            tok_table_hbm.at[idx_t.at[pl.ds(g * _CHUNK, _CHUNK)]],
            toks[b], tsems[b])
        pltpu.async_copy(
            combo_hbm.at[idx_c.at[pl.ds(g * _CHUNK, _CHUNK)]],
            cmbs[b], csems[b])

    def wait_gathers(b):
        pltpu.make_async_copy(
            tok_table_hbm.at[idx_t.at[pl.ds(0, _CHUNK)]],
            toks[b], tsems[b]).wait()
        pltpu.make_async_copy(
            combo_hbm.at[idx_c.at[pl.ds(0, _CHUNK)]],
            cmbs[b], csems[b]).wait()

    def wait_out(b):
        pltpu.make_async_copy(
            outs[b], out_hbm.at[pl.ds(0, _CHUNK)], osems[b]).wait()

    def start_out(g, b):
        pltpu.async_copy(
            outs[b], out_hbm.at[pl.ds(base + g * _CHUNK, _CHUNK)], osems[b])

    def compute(b):
        tbuf, cbuf, obuf = toks[b], cmbs[b], outs[b]

        def row_body(j, _):
            for k in range(D // 32):
                cw = cbuf[j, pl.ds(k * 16, 16)]
                cpair = plsc.bitcast(cw, jnp.bfloat16)
                ca, cb = plsc.unpack(cpair, format=plsc.PackFormat.INTERLEAVED)
                sa = pl.ds(k * 32, 16)
                sb = pl.ds(k * 32 + 16, 16)
                obuf[j, sa] = tbuf[j, sa] + ca
                obuf[j, sb] = tbuf[j, sb] + cb
            return ()

        lax.fori_loop(0, _CHUNK, row_body, ())

    start_gathers(0, 0)

    def pair_body(p, _):
        for b in range(2):
            g = p * 2 + b
            b1 = 1 - b
            if b == 0:
                start_gathers(g + 1, b1)
            else:
                @pl.when(p < _NCHUNK // 2 - 1)
                def _():
                    start_gathers(g + 1, b1)
            wait_gathers(b)

            @pl.when(p >= 1)
            def _():
                wait_out(b)
            compute(b)
            start_out(g, b)
        return ()

    lax.fori_loop(0, _NCHUNK // 2, pair_body, ())
    for b in range(2):
        wait_out(b)


@jax.jit
def kernel(sequence, segment_label, token_table, seg_table):
    tok_idx = sequence.reshape(-1).astype(jnp.int32)
    pos = jnp.arange(L, dtype=jnp.int32)
    combo_idx = (segment_label.astype(jnp.int32) * L + pos[None, :]).reshape(-1)
    combo = (seg_table[:, None, :] + jnp.asarray(_PE)[None, :, :]).reshape(3 * L, D)
    combo = (combo.reshape(3 * L, D // 32, 2, 16)
             .transpose(0, 1, 3, 2).reshape(3 * L, D // 2, 2)
             .astype(jnp.bfloat16))
    combo = lax.bitcast_convert_type(combo, jnp.int32)

    mesh = plsc.VectorSubcoreMesh(core_axis_name="c", subcore_axis_name="s")
    run = pl.kernel(
        _embed_kernel,
        mesh=mesh,
        compiler_params=pltpu.CompilerParams(
            needs_layout_passes=False, use_tc_tiling_on_sc=False),
        out_type=jax.ShapeDtypeStruct((_ROWS, D), jnp.float32),
        scratch_types=[
            pltpu.VMEM((_ROWS_PER_W,), jnp.int32),
            pltpu.VMEM((_ROWS_PER_W,), jnp.int32),
            pltpu.VMEM((_CHUNK, D), jnp.float32),
            pltpu.VMEM((_CHUNK, D), jnp.float32),
            pltpu.VMEM((_CHUNK, D // 2), jnp.int32),
            pltpu.VMEM((_CHUNK, D // 2), jnp.int32),
            pltpu.VMEM((_CHUNK, D), jnp.float32),
            pltpu.VMEM((_CHUNK, D), jnp.float32),
            pltpu.SemaphoreType.DMA,
            pltpu.SemaphoreType.DMA,
            pltpu.SemaphoreType.DMA,
            pltpu.SemaphoreType.DMA,
            pltpu.SemaphoreType.DMA,
            pltpu.SemaphoreType.DMA,
        ],
    )
    out = run(tok_idx, combo_idx, token_table, combo)
    return out.reshape(B, L, D)
